# baseline (speedup 1.0000x reference)
import functools
import math

import jax
import jax.numpy as jnp
from jax import lax
from jax.experimental import pallas as pl
from jax.experimental.pallas import tpu as pltpu
from jax.experimental.pallas import tpu_sc as plsc

VOCAB = 1_000_000
DIM = 64
SCALE = math.sqrt(DIM)

NC = 2
NS = 16
NW = NC * NS

B1 = 4096
B2 = 200
CHUNK = 128
NB = 2
G = 2
NSLOT = B2 // G


def _emb_body(table_h, xt_h, out_h, idx_v, rows_v, tbuf_v, *sems):
    wid = lax.axis_index("s") * NC + lax.axis_index("c")

    pltpu.sync_copy(xt_h.at[:, pl.ds(wid * CHUNK, CHUNK)], idx_v)

    gsems = sems[:NB]
    osems = sems[NB:]

    def gather_desc(j, b, gi):
        return pltpu.make_async_copy(
            table_h.at[idx_v.at[j * G + gi]], rows_v.at[b, gi], gsems[b])

    class _GatherGroup:
        def __init__(self, j, b):
            self.j, self.b = j, b

        def start(self):
            for gi in range(G):
                gather_desc(self.j, self.b, gi).start()

        def wait(self):
            for gi in range(G):
                gather_desc(self.j, self.b, gi).wait()

    def gather_copy(j, b):
        return _GatherGroup(j, b)

    def out_copy(j, b):
        return pltpu.make_async_copy(
            tbuf_v.at[b, :, :, :, pl.ds(0, CHUNK)],
            out_h.at[pl.ds(j * G, G), :, wid], osems[b])

    for b in range(NB):
        gather_copy(b, b).start()

    lanes = lax.iota(jnp.int32, 16)
    dts = [(16 * g + lanes) // 8 for g in range(DIM // 16)]
    dss = [(16 * g + lanes) % 8 for g in range(DIM // 16)]

    def chunk_step(t, carry):
        jj = t * NB
        for b in range(NB):
            j = jj + b
            gather_copy(j, b).wait()

            @pl.when(j >= NB)
            def _wait_prev_out():
                out_copy(j - NB, b).wait()

            for gi in range(G):
                rows = rows_v.at[b, gi]
                tbuf = tbuf_v.at[b, gi]

                def trans_s(s, c):
                    svec = lanes * 0 + s
                    for g in range(DIM // 16):
                        vec = rows[s, pl.ds(16 * g, 16)] * jnp.float32(SCALE)
                        plsc.store_scatter(tbuf, [dts[g], dss[g], svec], vec)
                    return c

                lax.fori_loop(0, CHUNK, trans_s, 0, unroll=4)

            @pl.when(j + NB < NSLOT)
            def _issue_next_gather():
                gather_copy(j + NB, b).start()

            out_copy(j, b).start()
        return carry

    lax.fori_loop(0, NSLOT // NB, chunk_step, 0)

    for b in range(NB):
        out_copy(NSLOT - NB + b, b).wait()


@jax.jit
def _emb_call(x_t, table):
    mesh = plsc.VectorSubcoreMesh(core_axis_name="c", subcore_axis_name="s")
    kfn = pl.kernel(
        _emb_body,
        out_type=jax.ShapeDtypeStruct((B2, 8, NW, 8, CHUNK), jnp.float32),
        mesh=mesh,
        compiler_params=pltpu.CompilerParams(
            use_tc_tiling_on_sc=False, needs_layout_passes=False),
        scratch_types=[
            pltpu.VMEM((B2, CHUNK), jnp.int32),
            pltpu.VMEM((NB, G, CHUNK, DIM), jnp.float32),
            pltpu.VMEM((NB, G, 8, 8, CHUNK + 1), jnp.float32),
        ] + [pltpu.SemaphoreType.DMA] * (2 * NB),
    )
    out5 = kfn(table, x_t)
    return out5.transpose(2, 4, 0, 1, 3).reshape(B1, B2, DIM)


def kernel(x, table):
    x_t = x.T.astype(jnp.int32)
    return _emb_call(x_t, table)


# speedup vs baseline: 1.0025x; 1.0025x over previous
import functools
import math

import jax
import jax.numpy as jnp
from jax import lax
from jax.experimental import pallas as pl
from jax.experimental.pallas import tpu as pltpu
from jax.experimental.pallas import tpu_sc as plsc

VOCAB = 1_000_000
DIM = 64
SCALE = math.sqrt(DIM)

NC = 2
NS = 16
NW = NC * NS

B1 = 4096
B2 = 200
CHUNK = 128
G = 2
NSLOT = B2 // G
NB = 2


def _emb_body(table_h, xidx_h, out_h, idx_v, rows_v, tbuf_v, *sems):
    wid = lax.axis_index("s") * NC + lax.axis_index("c")

    pltpu.sync_copy(xidx_h.at[:, wid], idx_v)

    gsems = sems[:NB]
    osems = sems[NB:]

    def gather_copy(j, b):
        return pltpu.make_async_copy(
            table_h.at[idx_v.at[j]], rows_v.at[b], gsems[b])

    def out_copy(j, b):
        return pltpu.make_async_copy(
            tbuf_v.at[b, :, :, :, pl.ds(0, CHUNK)],
            out_h.at[pl.ds(j * G, G), :, wid], osems[b])

    for b in range(NB):
        gather_copy(b, b).start()

    lanes = lax.iota(jnp.int32, 16)
    dts = [(16 * g + lanes) // 8 for g in range(DIM // 16)]
    dss = [(16 * g + lanes) % 8 for g in range(DIM // 16)]

    def chunk_step(t, carry):
        jj = t * NB
        for b in range(NB):
            j = jj + b
            gather_copy(j, b).wait()

            @pl.when(j >= NB)
            def _wait_prev_out():
                out_copy(j - NB, b).wait()

            rows = rows_v.at[b]
            for gi in range(G):
                tbuf = tbuf_v.at[b, gi]

                def trans_s(s, c):
                    r = gi * CHUNK + s
                    svec = lanes * 0 + s
                    for g in range(DIM // 16):
                        vec = rows[r, pl.ds(16 * g, 16)] * jnp.float32(SCALE)
                        plsc.store_scatter(tbuf, [dts[g], dss[g], svec], vec)
                    return c

                lax.fori_loop(0, CHUNK, trans_s, 0, unroll=4)

            @pl.when(j + NB < NSLOT)
            def _issue_next_gather():
                gather_copy(j + NB, b).start()

            out_copy(j, b).start()
        return carry

    lax.fori_loop(0, NSLOT // NB, chunk_step, 0)

    for b in range(NB):
        out_copy(NSLOT - NB + b, b).wait()


@jax.jit
def _emb_call(x_idx, table):
    mesh = plsc.VectorSubcoreMesh(core_axis_name="c", subcore_axis_name="s")
    kfn = pl.kernel(
        _emb_body,
        out_type=jax.ShapeDtypeStruct((B2, 8, NW, 8, CHUNK), jnp.float32),
        mesh=mesh,
        compiler_params=pltpu.CompilerParams(
            use_tc_tiling_on_sc=False, needs_layout_passes=False),
        scratch_types=[
            pltpu.VMEM((NSLOT, G * CHUNK), jnp.int32),
            pltpu.VMEM((NB, G * CHUNK, DIM), jnp.float32),
            pltpu.VMEM((NB, G, 8, 8, CHUNK + 1), jnp.float32),
        ] + [pltpu.SemaphoreType.DMA] * (2 * NB),
    )
    out5 = kfn(table, x_idx)
    return out5.transpose(2, 4, 0, 1, 3).reshape(B1, B2, DIM)


def kernel(x, table):
    x_idx = (x.T.astype(jnp.int32)
             .reshape(NSLOT, G, NW, CHUNK)
             .transpose(0, 2, 1, 3)
             .reshape(NSLOT, NW, G * CHUNK))
    return _emb_call(x_idx, table)


# speedup vs baseline: 1.4969x; 1.4932x over previous
import functools
import math

import jax
import jax.numpy as jnp
from jax import lax
from jax.experimental import pallas as pl
from jax.experimental.pallas import tpu as pltpu
from jax.experimental.pallas import tpu_sc as plsc

VOCAB = 1_000_000
DIM = 64
SCALE = math.sqrt(DIM)

NC = 2
NS = 16
NW = NC * NS

B1 = 4096
B2 = 200
CHUNK = 128
G = 2
NSLOT = B2 // G
NB = 2


def _emb_body(table_h, xidx_h, out_h, idx_v, rows_v, tbuf_v, *sems):
    wid = lax.axis_index("s") * NC + lax.axis_index("c")

    pltpu.sync_copy(xidx_h.at[:, wid], idx_v)

    gsems = sems[:NB]
    osems = sems[NB:]

    def gather_copy(j, b):
        return pltpu.make_async_copy(
            table_h.at[idx_v.at[j]], rows_v.at[b], gsems[b])

    def out_copy(j, b):
        return pltpu.make_async_copy(
            tbuf_v.at[b, :, :, :, pl.ds(0, CHUNK)],
            out_h.at[pl.ds(j * G, G), :, wid], osems[b])

    for b in range(NB):
        gather_copy(b, b).start()

    lanes = lax.iota(jnp.int32, 16)
    dts = [(16 * g + lanes) // 8 for g in range(DIM // 16)]
    dss = [(16 * g + lanes) % 8 for g in range(DIM // 16)]

    def chunk_step(t, carry):
        jj = t * NB
        for b in range(NB):
            j = jj + b
            gather_copy(j, b).wait()

            @pl.when(j >= NB)
            def _wait_prev_out():
                out_copy(j - NB, b).wait()

            rows = rows_v.at[b]
            for gi in range(G):
                tbuf = tbuf_v.at[b, gi]

                @plsc.parallel_loop(0, CHUNK, unroll=4)
                def trans_s(s):
                    r = gi * CHUNK + s
                    svec = lanes * 0 + s
                    for g in range(DIM // 16):
                        vec = rows[r, pl.ds(16 * g, 16)] * jnp.float32(SCALE)
                        plsc.store_scatter(tbuf, [dts[g], dss[g], svec], vec)

            @pl.when(j + NB < NSLOT)
            def _issue_next_gather():
                gather_copy(j + NB, b).start()

            out_copy(j, b).start()
        return carry

    lax.fori_loop(0, NSLOT // NB, chunk_step, 0)

    for b in range(NB):
        out_copy(NSLOT - NB + b, b).wait()


@jax.jit
def _emb_call(x_idx, table):
    mesh = plsc.VectorSubcoreMesh(core_axis_name="c", subcore_axis_name="s")
    kfn = pl.kernel(
        _emb_body,
        out_type=jax.ShapeDtypeStruct((B2, 8, NW, 8, CHUNK), jnp.float32),
        mesh=mesh,
        compiler_params=pltpu.CompilerParams(
            use_tc_tiling_on_sc=False, needs_layout_passes=False),
        scratch_types=[
            pltpu.VMEM((NSLOT, G * CHUNK), jnp.int32),
            pltpu.VMEM((NB, G * CHUNK, DIM), jnp.float32),
            pltpu.VMEM((NB, G, 8, 8, CHUNK + 1), jnp.float32),
        ] + [pltpu.SemaphoreType.DMA] * (2 * NB),
    )
    out5 = kfn(table, x_idx)
    return out5.transpose(2, 4, 0, 1, 3).reshape(B1, B2, DIM)


def kernel(x, table):
    x_idx = (x.T.astype(jnp.int32)
             .reshape(NSLOT, G, NW, CHUNK)
             .transpose(0, 2, 1, 3)
             .reshape(NSLOT, NW, G * CHUNK))
    return _emb_call(x_idx, table)
